# baseline (speedup 1.0000x reference)
import functools

import jax
import jax.numpy as jnp
from jax import lax
from jax.experimental import pallas as pl
from jax.experimental.pallas import tpu as pltpu
from jax.experimental.pallas import tpu_sc as plsc

N_FACTORS = 32
BATCH = 16384
TCOL = 128
HPAD = BATCH + 16
FLUSH = 32
GROWS = BATCH + 64


def _phase1(n_rows):
    info = plsc.get_sparse_core_info()
    nc, ns, nl = info.num_cores, info.num_subcores, info.num_lanes
    nw = nc * ns
    ntc_tot = (n_rows + TCOL - 1) // TCOL
    tc_per_w = (ntc_tot + nw - 1) // nw
    last_col = (n_rows // TCOL) * TCOL
    last_w = n_rows - last_col
    last_full = n_rows // TCOL - 1

    mesh = plsc.VectorSubcoreMesh(core_axis_name="c", subcore_axis_name="s")
    gshape = jax.ShapeDtypeStruct((GROWS, TCOL), jnp.float32)

    @functools.partial(
        pl.kernel,
        out_type=(gshape, gshape),
        mesh=mesh,
        compiler_params=pltpu.CompilerParams(
            needs_layout_passes=False, use_tc_tiling_on_sc=True),
        scratch_types=[
            pltpu.VMEM((HPAD,), jnp.int32),
            pltpu.VMEM((HPAD,), jnp.int32),
            pltpu.VMEM((HPAD,), jnp.int32),
            pltpu.VMEM((HPAD,), jnp.int32),
            pltpu.VMEM((256,), jnp.int32),
            pltpu.VMEM((272,), jnp.int32),
            pltpu.VMEM((256,), jnp.int32),
            pltpu.VMEM((12, N_FACTORS, TCOL), jnp.float32),
            pltpu.VMEM((N_FACTORS, 64), jnp.float32),
            pltpu.VMEM((2, FLUSH, TCOL), jnp.float32),
            pltpu.VMEM((2, FLUSH), jnp.int32),
            pltpu.SemaphoreType.DMA,
            pltpu.SemaphoreType.DMA,
            pltpu.SemaphoreType.DMA,
            pltpu.SemaphoreType.DMA,
            pltpu.SemaphoreType.DMA,
            pltpu.SemaphoreType.DMA,
            pltpu.SemaphoreType.DMA,
            pltpu.SemaphoreType.DMA,
            pltpu.SemaphoreType.DMA,
            pltpu.SemaphoreType.DMA,
            pltpu.SemaphoreType.DMA,
            pltpu.SemaphoreType.DMA,
            pltpu.SemaphoreType.DMA,
            pltpu.SemaphoreType.DMA,
        ],
    )
    def k(uidx_hbm, iidx_hbm, utabT_hbm, itabT_hbm, gu_hbm, gi_hbm,
          allidx, hval, hpos, spk, cnt_v, starts_v, cur_v,
          tiles, part_v, frows, fpos, st0, st1, st2, st3, st4, st5, st6, st7,
          st8, st9, st10, st11, semf, semp):
        wid = lax.axis_index("s") * nc + lax.axis_index("c")
        lanes = lax.iota(jnp.int32, nl)
        lane0 = lanes == 0
        tsems = (st0, st1, st2, st3, st4, st5, st6, st7, st8, st9, st10, st11)
        dump = BATCH + wid
        lo_tc = wid * tc_per_w
        hi_tc = jnp.minimum(lo_tc + tc_per_w, ntc_tot)
        ntc = hi_tc - lo_tc
        lo = lo_tc * TCOL
        hi = jnp.minimum(hi_tc * TCOL, n_rows)

        def sget(ref, i):
            return plsc.load_gather(ref, [jnp.full((nl,), i, jnp.int32)])[0]

        def sput(ref, i, v):
            plsc.store_scatter(ref, [jnp.full((nl,), i, jnp.int32)],
                               jnp.full((nl,), v), mask=lane0)

        def fetch_tile(tab_hbm, slot, tc, sem):
            cb = jnp.minimum(tc, last_full) * TCOL
            cb = pl.multiple_of(cb, TCOL)
            pltpu.async_copy(tab_hbm.at[:, pl.ds(cb, TCOL)],
                             tiles.at[slot], sem)

        def wait_tile(tab_hbm, slot, sem):
            pltpu.make_async_copy(tab_hbm.at[:, pl.ds(0, TCOL)],
                                  tiles.at[slot], sem).wait()

        def clear_fpos(b):
            fpos[b, pl.ds(0, nl)] = jnp.full((nl,), dump, jnp.int32)
            fpos[b, pl.ds(nl, nl)] = jnp.full((nl,), dump, jnp.int32)

        def run_table(idx_hbm, tab_hbm, out_hbm):
            pltpu.sync_copy(idx_hbm, allidx.at[pl.ds(0, BATCH)])
            allidx[pl.ds(BATCH, nl)] = jnp.full((nl,), 1 << 30, jnp.int32)
            pltpu.async_copy(tab_hbm.at[:, pl.ds(last_col, last_w)],
                             part_v, semp).wait()

            def scan_body(q, n):
                iv = allidx[pl.ds(q * nl, nl)]
                m = (iv >= lo) & (iv < hi)
                plsc.store_compressed(hval.at[pl.ds(n, nl)], iv, mask=m)
                plsc.store_compressed(hpos.at[pl.ds(n, nl)],
                                      q * nl + lanes, mask=m)
                return n + plsc.all_reduce_population_count(m)[0]

            n_hits = lax.fori_loop(0, BATCH // nl, scan_body, jnp.int32(0))
            nq = (n_hits + nl - 1) // nl

            for c in range(16):
                cnt_v[pl.ds(c * nl, nl)] = jnp.zeros((nl,), jnp.int32)

            def cnt_body(q, _):
                hv = hval[pl.ds(q * nl, nl)]
                for l in range(nl):
                    valid = q * nl + l < n_hits
                    b = jnp.where(valid, (hv[l] - lo) >> 7, 255)
                    sput(cnt_v, b, sget(cnt_v, b) + 1)
                return 0

            lax.fori_loop(0, nq, cnt_body, 0)

            def scan16(x):
                for s in (1, 2, 4, 8):
                    g = x.at[jnp.maximum(lanes - s, 0)].get(
                        mode="promise_in_bounds")
                    x = x + jnp.where(lanes >= s, g, 0)
                return x

            carry = jnp.int32(0)
            for c in range(16):
                v = cnt_v[pl.ds(c * nl, nl)]
                incl = scan16(v)
                excl = incl - v + carry
                starts_v[pl.ds(c * nl, nl)] = excl
                cur_v[pl.ds(c * nl, nl)] = excl
                carry = carry + incl[nl - 1]

            def place_body(q, _):
                hv = hval[pl.ds(q * nl, nl)]
                pv = hpos[pl.ds(q * nl, nl)]
                for l in range(nl):
                    valid = q * nl + l < n_hits
                    b = jnp.where(valid, (hv[l] - lo) >> 7, 255)
                    o = sget(cur_v, b)
                    ispart = (hv[l] >= last_col).astype(jnp.int32)
                    pclamp = jnp.clip(hv[l] - last_col, 0, last_w - 1)
                    pk = ((hv[l] & (TCOL - 1)) | (ispart << 7)
                          | (pclamp << 8) | (pv[l] << 14))
                    sput(spk, o, pk)
                    sput(cur_v, b, o + 1)
                return 0

            lax.fori_loop(0, nq, place_body, 0)

            clear_fpos(0)
            clear_fpos(1)
            for j in range(12):
                fetch_tile(tab_hbm, j, lo_tc + j, tsems[j])

            def make_hit_body(j):
                def hit_body(h, fc):
                    @pl.when((lax.rem(fc, FLUSH) == 0) & (fc >= 2 * FLUSH))
                    def _():
                        pltpu.make_async_copy(
                            frows.at[0], out_hbm.at[fpos.at[0]], semf).wait()
                        clear_fpos(lax.rem(fc // FLUSH, 2))

                    pk = sget(spk, h)
                    p = pk >> 14
                    cvec = jnp.full((nl,), pk & (TCOL - 1), jnp.int32)
                    pvec = jnp.full((nl,), (pk >> 8) & 63, jnp.int32)
                    ispart = ((pk >> 7) & 1) == 1
                    slotv = jnp.full((nl,), j, jnp.int32)
                    c_lo = jnp.where(
                        ispart,
                        plsc.load_gather(part_v, [lanes, pvec]),
                        plsc.load_gather(tiles, [slotv, lanes, cvec]))
                    c_hi = jnp.where(
                        ispart,
                        plsc.load_gather(part_v, [lanes + nl, pvec]),
                        plsc.load_gather(tiles, [slotv, lanes + nl, cvec]))
                    fb = lax.rem(fc // FLUSH, 2)
                    fs = lax.rem(fc, FLUSH)
                    frows[fb, fs, pl.ds(0, nl)] = c_lo
                    frows[fb, fs, pl.ds(nl, nl)] = c_hi
                    sput(fpos.at[fb], fs, p)
                    fc = fc + 1

                    @pl.when(lax.rem(fc, FLUSH) == 0)
                    def _():
                        fbd = lax.rem((fc - 1) // FLUSH, 2)
                        pltpu.async_copy(frows.at[fbd],
                                         out_hbm.at[fpos.at[fbd]], semf)

                    return fc
                return hit_body

            def oct_body(tq, fc):
                for j in range(12):
                    rel = tq * 12 + j
                    wait_tile(tab_hbm, j, tsems[j])
                    s0 = sget(starts_v, jnp.minimum(rel, ntc))
                    s1 = sget(starts_v, jnp.minimum(rel + 1, ntc))
                    s1 = jnp.where(rel < ntc, s1, s0)
                    fc = lax.fori_loop(s0, s1, make_hit_body(j), fc)
                    fetch_tile(tab_hbm, j, lo_tc + rel + 12, tsems[j])
                return fc

            noct = (ntc + 11) // 12
            fc = lax.fori_loop(0, noct, oct_body, jnp.int32(0))

            @pl.when(lax.rem(fc, FLUSH) != 0)
            def _():
                fb = lax.rem(fc // FLUSH, 2)
                pltpu.async_copy(frows.at[fb], out_hbm.at[fpos.at[fb]], semf)

            ndrain = jnp.minimum((fc + FLUSH - 1) // FLUSH, 2)

            def drain(i, _):
                pltpu.make_async_copy(frows.at[0], out_hbm.at[fpos.at[0]],
                                      semf).wait()
                return 0

            lax.fori_loop(0, ndrain, drain, 0)

            for j in range(12):
                wait_tile(tab_hbm, j, tsems[j])

        run_table(uidx_hbm, utabT_hbm, gu_hbm)
        run_table(iidx_hbm, itabT_hbm, gi_hbm)

    return k


def _phase2():
    info = plsc.get_sparse_core_info()
    nc, ns, nl = info.num_cores, info.num_subcores, info.num_lanes
    nw = nc * ns
    bpw = BATCH // nw
    nch = bpw // TCOL

    mesh = plsc.VectorSubcoreMesh(core_axis_name="c", subcore_axis_name="s")

    @functools.partial(
        pl.kernel,
        out_type=jax.ShapeDtypeStruct((BATCH,), jnp.float32),
        mesh=mesh,
        compiler_params=pltpu.CompilerParams(
            needs_layout_passes=False, use_tc_tiling_on_sc=True),
        scratch_types=[
            pltpu.VMEM((3, TCOL, TCOL), jnp.float32),
            pltpu.VMEM((3, TCOL, TCOL), jnp.float32),
            pltpu.VMEM((bpw,), jnp.float32),
            pltpu.VMEM((bpw,), jnp.float32),
            pltpu.VMEM((2 * N_FACTORS,), jnp.float32),
            pltpu.VMEM((N_FACTORS,), jnp.float32),
            pltpu.VMEM((N_FACTORS,), jnp.float32),
            pltpu.VMEM((N_FACTORS * nl,), jnp.float32),
            pltpu.VMEM((nl,), jnp.float32),
            pltpu.VMEM((bpw,), jnp.float32),
            pltpu.SemaphoreType.DMA,
            pltpu.SemaphoreType.DMA,
            pltpu.SemaphoreType.DMA,
        ],
    )
    def k(gu_hbm, gi_hbm, ctx0_hbm, ctx1_hbm, wc_hbm, bc_hbm, wo_hbm,
          wob_hbm, bo_hbm, out_hbm, ubuf, ibuf, ctx0_v, ctx1_v, wc_v,
          bc_v, wo_v, wob_v, bo_v, out_v, semA, semB, semC):
        wid = lax.axis_index("s") * nc + lax.axis_index("c")
        base = wid * bpw
        sems = (semA, semB, semC)
        lanes = lax.iota(jnp.int32, nl)

        pltpu.sync_copy(ctx0_hbm.at[pl.ds(base, bpw)], ctx0_v)
        pltpu.sync_copy(ctx1_hbm.at[pl.ds(base, bpw)], ctx1_v)
        pltpu.sync_copy(wc_hbm, wc_v)
        pltpu.sync_copy(bc_hbm, bc_v)
        pltpu.sync_copy(wo_hbm, wo_v)
        pltpu.sync_copy(wob_hbm, wob_v)
        pltpu.sync_copy(bo_hbm, bo_v)

        def allsum(x):
            for s in (8, 4, 2, 1):
                x = x + x.at[lanes ^ s].get(mode="promise_in_bounds")
            return x

        wo_lo = wo_v[pl.ds(0, nl)]
        wo_hi = wo_v[pl.ds(nl, nl)]
        wa = allsum(wc_v[pl.ds(0, nl)] * wo_lo + wc_v[pl.ds(nl, nl)] * wo_hi)
        wb = allsum(wc_v[pl.ds(2 * nl, nl)] * wo_lo
                    + wc_v[pl.ds(3 * nl, nl)] * wo_hi)
        const = (allsum(bc_v[pl.ds(0, nl)] * wo_lo
                        + bc_v[pl.ds(nl, nl)] * wo_hi)
                 + bo_v[...])

        def enqueue(c, par):
            pltpu.async_copy(gu_hbm.at[pl.ds(base + c * TCOL, TCOL)],
                             ubuf.at[par], sems[par])
            pltpu.async_copy(gi_hbm.at[pl.ds(base + c * TCOL, TCOL)],
                             ibuf.at[par], sems[par])

        def wait(par):
            pltpu.make_async_copy(gu_hbm.at[pl.ds(0, TCOL)],
                                  ubuf.at[par], sems[par]).wait()
            pltpu.make_async_copy(gi_hbm.at[pl.ds(0, TCOL)],
                                  ibuf.at[par], sems[par]).wait()

        enqueue(0, 0)
        enqueue(1, 1)
        for c in range(nch):
            par = c % 3
            if c + 2 < nch:
                enqueue(c + 2, (c + 2) % 3)
            wait(par)
            pv = jnp.full((nl,), par, jnp.int32)
            for g in range(TCOL // nl):
                row = g * nl + lanes
                e0 = c * TCOL + g * nl
                acc = (ctx0_v[pl.ds(e0, nl)] * wa
                       + ctx1_v[pl.ds(e0, nl)] * wb + const)
                for f in range(N_FACTORS):
                    col = jnp.full((nl,), f, jnp.int32)
                    u = plsc.load_gather(ubuf, [pv, row, col])
                    v = plsc.load_gather(ibuf, [pv, row, col])
                    wf = wob_v[pl.ds(f * nl, nl)]
                    acc = acc + u * v * wf
                out_v[pl.ds(e0, nl)] = acc
        pltpu.sync_copy(out_v, out_hbm.at[pl.ds(base, bpw)])

    return k


def kernel(user, item, context, user_table, item_table, Wc, bc, Wo, bo):
    p1 = _phase1(user_table.shape[0])
    p2 = _phase2()
    user_i = user.astype(jnp.int32)
    item_i = item.astype(jnp.int32)
    ctx0 = context[:, 0]
    ctx1 = context[:, 1]
    wc_flat = Wc.reshape(2 * N_FACTORS)
    wo_flat = Wo.reshape(N_FACTORS)
    wob_flat = jnp.broadcast_to(
        Wo.reshape(N_FACTORS, 1), (N_FACTORS, 16)).reshape(N_FACTORS * 16)
    bo_splat = jnp.broadcast_to(bo, (16,))
    gu, gi = p1(user_i, item_i, user_table.T, item_table.T)
    return p2(gu, gi, ctx0, ctx1, wc_flat, bc, wo_flat, wob_flat, bo_splat)


# speedup vs baseline: 1.0740x; 1.0740x over previous
import functools

import jax
import jax.numpy as jnp
from jax import lax
from jax.experimental import pallas as pl
from jax.experimental.pallas import tpu as pltpu
from jax.experimental.pallas import tpu_sc as plsc

N_FACTORS = 32
BATCH = 16384
TCOL = 128
HPAD = BATCH + 16
FLUSH = 32
GROWS = BATCH + 64


def _phase1(n_rows):
    info = plsc.get_sparse_core_info()
    nc, ns, nl = info.num_cores, info.num_subcores, info.num_lanes
    nw = nc * ns
    ntc_tot = (n_rows + TCOL - 1) // TCOL
    tc_per_w = (ntc_tot + nw - 1) // nw
    last_col = (n_rows // TCOL) * TCOL
    last_w = n_rows - last_col
    last_full = n_rows // TCOL - 1

    mesh = plsc.VectorSubcoreMesh(core_axis_name="c", subcore_axis_name="s")
    gshape = jax.ShapeDtypeStruct((GROWS, TCOL), jnp.float32)

    @functools.partial(
        pl.kernel,
        out_type=(gshape, gshape),
        mesh=mesh,
        compiler_params=pltpu.CompilerParams(
            needs_layout_passes=False, use_tc_tiling_on_sc=True),
        scratch_types=[
            pltpu.VMEM((HPAD,), jnp.int32),
            pltpu.VMEM((HPAD,), jnp.int32),
            pltpu.VMEM((HPAD,), jnp.int32),
            pltpu.VMEM((HPAD,), jnp.int32),
            pltpu.VMEM((256,), jnp.int32),
            pltpu.VMEM((272,), jnp.int32),
            pltpu.VMEM((256,), jnp.int32),
            pltpu.VMEM((8, N_FACTORS, TCOL), jnp.float32),
            pltpu.VMEM((N_FACTORS, 64), jnp.float32),
            pltpu.VMEM((2, FLUSH, TCOL), jnp.float32),
            pltpu.VMEM((2, FLUSH), jnp.int32),
            pltpu.SemaphoreType.DMA,
            pltpu.SemaphoreType.DMA,
            pltpu.SemaphoreType.DMA,
            pltpu.SemaphoreType.DMA,
            pltpu.SemaphoreType.DMA,
            pltpu.SemaphoreType.DMA,
            pltpu.SemaphoreType.DMA,
            pltpu.SemaphoreType.DMA,
            pltpu.SemaphoreType.DMA,
            pltpu.SemaphoreType.DMA,
        ],
    )
    def k(uidx_hbm, iidx_hbm, utabT_hbm, itabT_hbm, gu_hbm, gi_hbm,
          allidx, hval, hpos, spk, cnt_v, starts_v, cur_v,
          tiles, part_v, frows, fpos, st0, st1, st2, st3, st4, st5, st6, st7,
          semf, semp):
        wid = lax.axis_index("s") * nc + lax.axis_index("c")
        lanes = lax.iota(jnp.int32, nl)
        lane0 = lanes == 0
        tsems = (st0, st1, st2, st3, st4, st5, st6, st7)
        dump = BATCH + wid
        lo_tc = wid * tc_per_w
        hi_tc = jnp.minimum(lo_tc + tc_per_w, ntc_tot)
        ntc = hi_tc - lo_tc
        lo = lo_tc * TCOL
        hi = jnp.minimum(hi_tc * TCOL, n_rows)

        def sget(ref, i):
            return plsc.load_gather(ref, [jnp.full((nl,), i, jnp.int32)])[0]

        def sput(ref, i, v):
            plsc.store_scatter(ref, [jnp.full((nl,), i, jnp.int32)],
                               jnp.full((nl,), v), mask=lane0)

        def fetch_tile(tab_hbm, slot, tc, sem):
            cb = jnp.minimum(tc, last_full) * TCOL
            cb = pl.multiple_of(cb, TCOL)
            pltpu.async_copy(tab_hbm.at[:, pl.ds(cb, TCOL)],
                             tiles.at[slot], sem)

        def wait_tile(tab_hbm, slot, sem):
            pltpu.make_async_copy(tab_hbm.at[:, pl.ds(0, TCOL)],
                                  tiles.at[slot], sem).wait()

        def clear_fpos(b):
            fpos[b, pl.ds(0, nl)] = jnp.full((nl,), dump, jnp.int32)
            fpos[b, pl.ds(nl, nl)] = jnp.full((nl,), dump, jnp.int32)

        def run_table(idx_hbm, tab_hbm, out_hbm):
            pltpu.sync_copy(idx_hbm, allidx.at[pl.ds(0, BATCH)])
            allidx[pl.ds(BATCH, nl)] = jnp.full((nl,), 1 << 30, jnp.int32)
            pltpu.async_copy(tab_hbm.at[:, pl.ds(last_col, last_w)],
                             part_v, semp).wait()

            def scan_body(q, n):
                iv = allidx[pl.ds(q * nl, nl)]
                m = (iv >= lo) & (iv < hi)
                plsc.store_compressed(hval.at[pl.ds(n, nl)], iv, mask=m)
                plsc.store_compressed(hpos.at[pl.ds(n, nl)],
                                      q * nl + lanes, mask=m)
                return n + plsc.all_reduce_population_count(m)[0]

            n_hits = lax.fori_loop(0, BATCH // nl, scan_body, jnp.int32(0))
            nq = (n_hits + nl - 1) // nl

            for c in range(16):
                cnt_v[pl.ds(c * nl, nl)] = jnp.zeros((nl,), jnp.int32)

            def cnt_body(q, _):
                hv = hval[pl.ds(q * nl, nl)]
                for l in range(nl):
                    valid = q * nl + l < n_hits
                    b = jnp.where(valid, (hv[l] - lo) >> 7, 255)
                    sput(cnt_v, b, sget(cnt_v, b) + 1)
                return 0

            lax.fori_loop(0, nq, cnt_body, 0)

            def scan16(x):
                for s in (1, 2, 4, 8):
                    g = x.at[jnp.maximum(lanes - s, 0)].get(
                        mode="promise_in_bounds")
                    x = x + jnp.where(lanes >= s, g, 0)
                return x

            carry = jnp.int32(0)
            for c in range(16):
                v = cnt_v[pl.ds(c * nl, nl)]
                incl = scan16(v)
                excl = incl - v + carry
                starts_v[pl.ds(c * nl, nl)] = excl
                cur_v[pl.ds(c * nl, nl)] = excl
                carry = carry + incl[nl - 1]

            def place_body(q, _):
                hv = hval[pl.ds(q * nl, nl)]
                pv = hpos[pl.ds(q * nl, nl)]
                for l in range(nl):
                    valid = q * nl + l < n_hits
                    b = jnp.where(valid, (hv[l] - lo) >> 7, 255)
                    o = sget(cur_v, b)
                    ispart = (hv[l] >= last_col).astype(jnp.int32)
                    pclamp = jnp.clip(hv[l] - last_col, 0, last_w - 1)
                    pk = ((hv[l] & (TCOL - 1)) | (ispart << 7)
                          | (pclamp << 8) | (pv[l] << 14))
                    sput(spk, o, pk)
                    sput(cur_v, b, o + 1)
                return 0

            lax.fori_loop(0, nq, place_body, 0)

            clear_fpos(0)
            clear_fpos(1)
            for j in range(8):
                fetch_tile(tab_hbm, j, lo_tc + j, tsems[j])

            def make_hit_body(j):
                def hit_body(h, fc):
                    @pl.when((lax.rem(fc, FLUSH) == 0) & (fc >= 2 * FLUSH))
                    def _():
                        pltpu.make_async_copy(
                            frows.at[0], out_hbm.at[fpos.at[0]], semf).wait()
                        clear_fpos(lax.rem(fc // FLUSH, 2))

                    pk = sget(spk, h)
                    p = pk >> 14
                    cvec = jnp.full((nl,), pk & (TCOL - 1), jnp.int32)
                    pvec = jnp.full((nl,), (pk >> 8) & 63, jnp.int32)
                    ispart = ((pk >> 7) & 1) == 1
                    slotv = jnp.full((nl,), j, jnp.int32)
                    c_lo = jnp.where(
                        ispart,
                        plsc.load_gather(part_v, [lanes, pvec]),
                        plsc.load_gather(tiles, [slotv, lanes, cvec]))
                    c_hi = jnp.where(
                        ispart,
                        plsc.load_gather(part_v, [lanes + nl, pvec]),
                        plsc.load_gather(tiles, [slotv, lanes + nl, cvec]))
                    fb = lax.rem(fc // FLUSH, 2)
                    fs = lax.rem(fc, FLUSH)
                    frows[fb, fs, pl.ds(0, nl)] = c_lo
                    frows[fb, fs, pl.ds(nl, nl)] = c_hi
                    sput(fpos.at[fb], fs, p)
                    fc = fc + 1

                    @pl.when(lax.rem(fc, FLUSH) == 0)
                    def _():
                        fbd = lax.rem((fc - 1) // FLUSH, 2)
                        pltpu.async_copy(frows.at[fbd],
                                         out_hbm.at[fpos.at[fbd]], semf)

                    return fc
                return hit_body

            def oct_body(tq, fc):
                for j in range(8):
                    rel = tq * 8 + j
                    wait_tile(tab_hbm, j, tsems[j])
                    s0 = sget(starts_v, jnp.minimum(rel, ntc))
                    s1 = sget(starts_v, jnp.minimum(rel + 1, ntc))
                    s1 = jnp.where(rel < ntc, s1, s0)
                    fc = lax.fori_loop(s0, s1, make_hit_body(j), fc)
                    fetch_tile(tab_hbm, j, lo_tc + rel + 8, tsems[j])
                return fc

            noct = (ntc + 7) // 8
            fc = lax.fori_loop(0, noct, oct_body, jnp.int32(0))

            @pl.when(lax.rem(fc, FLUSH) != 0)
            def _():
                fb = lax.rem(fc // FLUSH, 2)
                pltpu.async_copy(frows.at[fb], out_hbm.at[fpos.at[fb]], semf)

            ndrain = jnp.minimum((fc + FLUSH - 1) // FLUSH, 2)

            def drain(i, _):
                pltpu.make_async_copy(frows.at[0], out_hbm.at[fpos.at[0]],
                                      semf).wait()
                return 0

            lax.fori_loop(0, ndrain, drain, 0)

            for j in range(8):
                wait_tile(tab_hbm, j, tsems[j])

        run_table(uidx_hbm, utabT_hbm, gu_hbm)
        run_table(iidx_hbm, itabT_hbm, gi_hbm)

    return k


def _phase2():
    info = plsc.get_sparse_core_info()
    nc, ns, nl = info.num_cores, info.num_subcores, info.num_lanes
    nw = nc * ns
    bpw = BATCH // nw
    nch = bpw // TCOL

    mesh = plsc.VectorSubcoreMesh(core_axis_name="c", subcore_axis_name="s")

    @functools.partial(
        pl.kernel,
        out_type=jax.ShapeDtypeStruct((BATCH,), jnp.float32),
        mesh=mesh,
        compiler_params=pltpu.CompilerParams(
            needs_layout_passes=False, use_tc_tiling_on_sc=True),
        scratch_types=[
            pltpu.VMEM((3, TCOL, TCOL), jnp.float32),
            pltpu.VMEM((3, TCOL, TCOL), jnp.float32),
            pltpu.VMEM((bpw,), jnp.float32),
            pltpu.VMEM((bpw,), jnp.float32),
            pltpu.VMEM((2 * N_FACTORS,), jnp.float32),
            pltpu.VMEM((N_FACTORS,), jnp.float32),
            pltpu.VMEM((N_FACTORS,), jnp.float32),
            pltpu.VMEM((N_FACTORS * nl,), jnp.float32),
            pltpu.VMEM((nl,), jnp.float32),
            pltpu.VMEM((bpw,), jnp.float32),
            pltpu.SemaphoreType.DMA,
            pltpu.SemaphoreType.DMA,
            pltpu.SemaphoreType.DMA,
        ],
    )
    def k(gu_hbm, gi_hbm, ctx0_hbm, ctx1_hbm, wc_hbm, bc_hbm, wo_hbm,
          wob_hbm, bo_hbm, out_hbm, ubuf, ibuf, ctx0_v, ctx1_v, wc_v,
          bc_v, wo_v, wob_v, bo_v, out_v, semA, semB, semC):
        wid = lax.axis_index("s") * nc + lax.axis_index("c")
        base = wid * bpw
        sems = (semA, semB, semC)
        lanes = lax.iota(jnp.int32, nl)

        pltpu.sync_copy(ctx0_hbm.at[pl.ds(base, bpw)], ctx0_v)
        pltpu.sync_copy(ctx1_hbm.at[pl.ds(base, bpw)], ctx1_v)
        pltpu.sync_copy(wc_hbm, wc_v)
        pltpu.sync_copy(bc_hbm, bc_v)
        pltpu.sync_copy(wo_hbm, wo_v)
        pltpu.sync_copy(wob_hbm, wob_v)
        pltpu.sync_copy(bo_hbm, bo_v)

        def allsum(x):
            for s in (8, 4, 2, 1):
                x = x + x.at[lanes ^ s].get(mode="promise_in_bounds")
            return x

        wo_lo = wo_v[pl.ds(0, nl)]
        wo_hi = wo_v[pl.ds(nl, nl)]
        wa = allsum(wc_v[pl.ds(0, nl)] * wo_lo + wc_v[pl.ds(nl, nl)] * wo_hi)
        wb = allsum(wc_v[pl.ds(2 * nl, nl)] * wo_lo
                    + wc_v[pl.ds(3 * nl, nl)] * wo_hi)
        const = (allsum(bc_v[pl.ds(0, nl)] * wo_lo
                        + bc_v[pl.ds(nl, nl)] * wo_hi)
                 + bo_v[...])

        def enqueue(c, par):
            pltpu.async_copy(gu_hbm.at[pl.ds(base + c * TCOL, TCOL)],
                             ubuf.at[par], sems[par])
            pltpu.async_copy(gi_hbm.at[pl.ds(base + c * TCOL, TCOL)],
                             ibuf.at[par], sems[par])

        def wait(par):
            pltpu.make_async_copy(gu_hbm.at[pl.ds(0, TCOL)],
                                  ubuf.at[par], sems[par]).wait()
            pltpu.make_async_copy(gi_hbm.at[pl.ds(0, TCOL)],
                                  ibuf.at[par], sems[par]).wait()

        enqueue(0, 0)
        enqueue(1, 1)
        for c in range(nch):
            par = c % 3
            if c + 2 < nch:
                enqueue(c + 2, (c + 2) % 3)
            wait(par)
            pv = jnp.full((nl,), par, jnp.int32)
            for g in range(TCOL // nl):
                row = g * nl + lanes
                e0 = c * TCOL + g * nl
                acc = (ctx0_v[pl.ds(e0, nl)] * wa
                       + ctx1_v[pl.ds(e0, nl)] * wb + const)
                for f in range(N_FACTORS):
                    col = jnp.full((nl,), f, jnp.int32)
                    u = plsc.load_gather(ubuf, [pv, row, col])
                    v = plsc.load_gather(ibuf, [pv, row, col])
                    wf = wob_v[pl.ds(f * nl, nl)]
                    acc = acc + u * v * wf
                out_v[pl.ds(e0, nl)] = acc
        pltpu.sync_copy(out_v, out_hbm.at[pl.ds(base, bpw)])

    return k


def kernel(user, item, context, user_table, item_table, Wc, bc, Wo, bo):
    p1 = _phase1(user_table.shape[0])
    p2 = _phase2()
    user_i = user.astype(jnp.int32)
    item_i = item.astype(jnp.int32)
    ctx0 = context[:, 0]
    ctx1 = context[:, 1]
    wc_flat = Wc.reshape(2 * N_FACTORS)
    wo_flat = Wo.reshape(N_FACTORS)
    wob_flat = jnp.broadcast_to(
        Wo.reshape(N_FACTORS, 1), (N_FACTORS, 16)).reshape(N_FACTORS * 16)
    bo_splat = jnp.broadcast_to(bo, (16,))
    gu, gi = p1(user_i, item_i, user_table.T, item_table.T)
    return p2(gu, gi, ctx0, ctx1, wc_flat, bc, wo_flat, wob_flat, bo_splat)
